# baseline (speedup 1.0000x reference)
import functools

import jax
import jax.numpy as jnp
from jax import lax
from jax.experimental import pallas as pl
from jax.experimental.pallas import tpu as pltpu
from jax.experimental.pallas import tpu_sc as plsc

F = 128
NC = 2
NS = 16
NW = NC * NS
L = 16
SB = 64


def _tc_prep_body(x_ref, w_ref, asd_ref, xt_ref, al_ref, stats_ref):
    xt = jnp.dot(x_ref[...], w_ref[...], preferred_element_type=jnp.float32)
    xt_ref[...] = xt
    al = jnp.dot(xt, asd_ref[...], preferred_element_type=jnp.float32)
    al_ref[...] = al
    m = jnp.maximum(jnp.max(al[:, 0]) + jnp.max(al[:, 1]), 0.0)
    stats_ref[...] = jnp.full((8, 128), m, jnp.float32)


def _tc_attr_body(ea_ref, b_ref, attr_ref):
    attr_ref[...] = jnp.dot(ea_ref[...], b_ref[...],
                            preferred_element_type=jnp.float32)


def _tc_div_body(n, op_ref, sp_ref, xt_ref, al_ref, stats_ref, o_ref):
    m = stats_ref[0, 0]
    lg = al_ref[:n, 0] + al_ref[:n, 1]
    lg = jnp.where(lg > 0.0, lg, 0.2 * lg)
    lee = jnp.exp(lg - m)
    num = op_ref[0, :n, :] + op_ref[1, :n, :] + xt_ref[:n, :] * lee[:, None]
    den = sp_ref[0, :n] + sp_ref[1, :n] + lee
    o_ref[...] = num / den[:, None]


def _gat_kernel_body(np_pad, npt,
                     src_hbm, dst_hbm, attr_hbm, as_hbm, ad_hbm, stats_hbm,
                     xt_hbm,
                     segpart_hbm, outpart_hbm,
                     as_v, ad_v, stat_v, zbuf, src_v, dst_v, attr_v, ee_v,
                     rows_v, shared_seg, shared_out,
                     sem_i0, sem_i1, sem_g0, sem_g1, sem_s0, sem_s1,
                     sem_e0, sem_e1):
    cid = lax.axis_index("c")
    sid = lax.axis_index("s")
    tid = cid * NS + sid
    sem_i = (sem_i0, sem_i1)
    sem_g = (sem_g0, sem_g1)
    sem_s = (sem_s0, sem_s1)
    sem_e = (sem_e0, sem_e1)
    base_r = tid * npt

    def issue_loads(r, s):
        pltpu.async_copy(src_hbm.at[r], src_v.at[s], sem_i[s])
        pltpu.async_copy(dst_hbm.at[r], dst_v.at[s], sem_i[s])
        pltpu.async_copy(attr_hbm.at[r], attr_v.at[s], sem_i[s])

    def wait_loads(s):
        pltpu.make_async_copy(src_hbm.at[0], src_v.at[s], sem_i[s]).wait()
        pltpu.make_async_copy(dst_hbm.at[0], dst_v.at[s], sem_i[s]).wait()
        pltpu.make_async_copy(attr_hbm.at[0], attr_v.at[s], sem_i[s]).wait()

    def gidx(s, rev):
        return dst_v.at[s] if rev else src_v.at[s]

    def sidx(s, rev):
        return src_v.at[s] if rev else dst_v.at[s]

    def issue_gather(s, rev, b):
        pltpu.async_copy(xt_hbm.at[gidx(s, rev)], rows_v.at[b], sem_g[b])

    def wait_gather(s, rev, b):
        pltpu.make_async_copy(
            xt_hbm.at[gidx(s, rev)], rows_v.at[b], sem_g[b]).wait()

    def issue_scatter(s, rev, b):
        pltpu.async_copy(rows_v.at[b], shared_out.at[sidx(s, rev)],
                         sem_s[b], add=True)

    def wait_scatter(s, rev, b):
        pltpu.make_async_copy(
            rows_v.at[b], shared_out.at[sidx(s, rev)], sem_s[b]).wait()

    def issue_escatter(s, rev, b):
        pltpu.async_copy(ee_v.at[b], shared_seg.at[sidx(s, rev)],
                         sem_e[b], add=True)

    def wait_escatter(s, rev, b):
        pltpu.make_async_copy(
            ee_v.at[b], shared_seg.at[sidx(s, rev)], sem_e[b]).wait()

    def compute(s, rev, b):
        def _grp(g, _):
            sl = pl.ds(g * L, L)
            si = src_v[s, sl]
            di = dst_v[s, sl]
            gi, sc = (di, si) if rev else (si, di)
            lg = (plsc.load_gather(as_v, [gi])
                  + plsc.load_gather(ad_v, [sc])
                  + attr_v[s, sl])
            lg = jnp.where(lg > 0.0, lg, 0.2 * lg)
            ee = jnp.exp(lg - m_vec)
            ee_v[b, sl] = ee
            for lane in range(L):
                aa = ee[lane]
                row = g * L + lane
                for k in range(F // L):
                    ck = pl.ds(k * L, L)
                    rows_v[b, row, ck] = rows_v[b, row, ck] * aa
            return 0
        lax.fori_loop(0, SB // L, _grp, 0)

    pltpu.sync_copy(as_hbm, as_v)
    pltpu.sync_copy(ad_hbm, ad_v)
    pltpu.sync_copy(stats_hbm.at[0], stat_v)
    m_vec = stat_v[pl.ds(0, L)]

    zero = jnp.zeros((L,), jnp.float32)
    seg_slice = np_pad // NS

    def _zseg(i, _):
        zbuf[pl.ds(i * L, L)] = zero
        return 0
    lax.fori_loop(0, seg_slice // L, _zseg, 0)
    pltpu.sync_copy(zbuf, shared_seg.at[pl.ds(sid * seg_slice, seg_slice)])

    def _zrow(i, _):
        def _zcol(k, _):
            rows_v[0, i, pl.ds(k * L, L)] = zero
            return 0
        lax.fori_loop(0, F // L, _zcol, 0)
        return 0
    lax.fori_loop(0, SB, _zrow, 0)
    rows_slice = np_pad // NS
    row0 = sid * rows_slice
    for r in range(rows_slice // SB):
        pltpu.sync_copy(rows_v.at[0], shared_out.at[pl.ds(row0 + r * SB, SB)])
    plsc.subcore_barrier()

    issue_loads(base_r, 0)
    issue_loads(base_r + 1, 1)
    wait_loads(0)
    issue_gather(0, False, 0)

    def _steady(k, _):
        p0 = k * 2

        wait_gather(0, False, 0)

        @pl.when(k >= 1)
        def _():
            wait_scatter(1, True, 1)
            wait_escatter(0, False, 0)
        issue_gather(0, True, 1)
        compute(0, False, 0)
        issue_escatter(0, False, 0)
        issue_scatter(0, False, 0)

        wait_gather(0, True, 1)
        wait_scatter(0, False, 0)

        @pl.when(k >= 1)
        def _():
            wait_escatter(1, True, 1)
        wait_loads(1)
        issue_gather(1, False, 0)
        compute(0, True, 1)
        issue_escatter(0, True, 1)
        issue_scatter(0, True, 1)

        @pl.when(p0 + 2 < npt)
        def _():
            issue_loads(base_r + p0 + 2, 0)

        wait_gather(1, False, 0)
        wait_scatter(0, True, 1)
        wait_escatter(0, False, 0)
        issue_gather(1, True, 1)
        compute(1, False, 0)
        issue_escatter(1, False, 0)
        issue_scatter(1, False, 0)

        wait_gather(1, True, 1)
        wait_scatter(1, False, 0)
        wait_escatter(0, True, 1)

        @pl.when(p0 + 2 < npt)
        def _():
            wait_loads(0)
            issue_gather(0, False, 0)
        compute(1, True, 1)
        issue_escatter(1, True, 1)
        issue_scatter(1, True, 1)

        @pl.when(p0 + 3 < npt)
        def _():
            issue_loads(base_r + p0 + 3, 1)
        return 0
    lax.fori_loop(0, npt // 2, _steady, 0)
    wait_scatter(1, True, 1)
    wait_escatter(1, False, 0)
    wait_escatter(1, True, 1)

    plsc.subcore_barrier()
    pltpu.sync_copy(shared_seg.at[pl.ds(sid * seg_slice, seg_slice)],
                    segpart_hbm.at[cid, pl.ds(sid * seg_slice, seg_slice)])
    pltpu.sync_copy(shared_out.at[pl.ds(row0, rows_slice)],
                    outpart_hbm.at[cid, pl.ds(row0, rows_slice)])


def kernel(x, edge_index, edge_attr, batch, W, a):
    n = x.shape[0]
    e = edge_attr.shape[0]
    ed = edge_attr.shape[1]
    np_pad = ((n + NS * L * 8 - 1) // (NS * L * 8)) * (NS * L * 8)
    npt = ((e + NW * SB * 2 - 1) // (NW * SB * 2)) * 2
    e_pad = npt * NW * SB

    src = edge_index[0].astype(jnp.int32)
    dst = edge_index[1].astype(jnp.int32)
    padi = jnp.full((e_pad - e,), n, dtype=jnp.int32)
    src_p = jnp.concatenate([src, padi])
    dst_p = jnp.concatenate([dst, padi])

    x_pad = jnp.pad(x, ((0, np_pad - n), (0, 0)))
    a_sd = a[:2 * F, 0].reshape(2, F).T

    xt_pad, alpha, stats = pl.pallas_call(
        _tc_prep_body,
        out_shape=(
            jax.ShapeDtypeStruct((np_pad, F), jnp.float32),
            jax.ShapeDtypeStruct((np_pad, 2), jnp.float32),
            jax.ShapeDtypeStruct((8, 128), jnp.float32),
        ),
    )(x_pad, W, a_sd)

    gp = 128 // ed
    ea128 = edge_attr.reshape(e // gp, 128)
    a3 = a[2 * F:, 0]
    b_blk = jnp.zeros((128, gp), jnp.float32)
    b_blk = b_blk.at[jnp.arange(128), jnp.arange(128) // ed].set(
        jnp.tile(a3, gp))
    attr8 = pl.pallas_call(
        _tc_attr_body,
        out_shape=jax.ShapeDtypeStruct((e // gp, gp), jnp.float32),
    )(ea128, b_blk)
    attr_p = jnp.concatenate(
        [attr8.reshape(e), jnp.zeros((e_pad - e,), jnp.float32)])

    src2d = src_p.reshape(e_pad // SB, SB)
    dst2d = dst_p.reshape(e_pad // SB, SB)
    attr2d = attr_p.reshape(e_pad // SB, SB)
    alpha_s = alpha[:, 0]
    alpha_d = alpha[:, 1]

    gat_kernel = pl.kernel(
        functools.partial(_gat_kernel_body, np_pad, npt),
        out_type=(
            jax.ShapeDtypeStruct((NC, np_pad), jnp.float32),
            jax.ShapeDtypeStruct((NC, np_pad, F), jnp.float32),
        ),
        mesh=plsc.VectorSubcoreMesh(core_axis_name="c", subcore_axis_name="s"),
        compiler_params=pltpu.CompilerParams(needs_layout_passes=False),
        scratch_types=(
            pltpu.VMEM((np_pad,), jnp.float32),
            pltpu.VMEM((np_pad,), jnp.float32),
            pltpu.VMEM((128,), jnp.float32),
            pltpu.VMEM((np_pad // NS,), jnp.float32),
            pltpu.VMEM((2, SB), jnp.int32),
            pltpu.VMEM((2, SB), jnp.int32),
            pltpu.VMEM((2, SB), jnp.float32),
            pltpu.VMEM((2, SB), jnp.float32),
            pltpu.VMEM((2, SB, F), jnp.float32),
            pltpu.VMEM_SHARED((np_pad,), jnp.float32),
            pltpu.VMEM_SHARED((np_pad, F), jnp.float32),
            pltpu.SemaphoreType.DMA,
            pltpu.SemaphoreType.DMA,
            pltpu.SemaphoreType.DMA,
            pltpu.SemaphoreType.DMA,
            pltpu.SemaphoreType.DMA,
            pltpu.SemaphoreType.DMA,
            pltpu.SemaphoreType.DMA,
            pltpu.SemaphoreType.DMA,
        ),
    )
    segpart, outpart = gat_kernel(src2d, dst2d, attr2d, alpha_s, alpha_d,
                                  stats, xt_pad)

    out = pl.pallas_call(
        functools.partial(_tc_div_body, n),
        out_shape=jax.ShapeDtypeStruct((n, F), jnp.float32),
    )(outpart, segpart, xt_pad, alpha, stats)
    return out
